# baseline (speedup 1.0000x reference)
import functools

import jax
import jax.numpy as jnp
from jax import lax
from jax.experimental import pallas as pl
from jax.experimental.pallas import tpu as pltpu
from jax.experimental.pallas import tpu_sc as plsc

NUM_ITEM = 1000000
NUM_FACTOR = 64
BATCH = 4096
TARGET_LEN = 20
NEG_LEN = 100
SEQ = TARGET_LEN + NEG_LEN
TOTAL = BATCH * SEQ

NC = 2
NS = 16
NW = NC * NS
G = 128
PER_W = TOTAL // NW
NG = PER_W // G
NBUF = 4
NSTEP = NG // NBUF

_mesh = plsc.VectorSubcoreMesh(
    core_axis_name="c", subcore_axis_name="s", num_cores=NC, num_subcores=NS)


@functools.partial(
    pl.kernel,
    out_type=jax.ShapeDtypeStruct((TOTAL, NUM_FACTOR), jnp.float32),
    mesh=_mesh,
    scratch_types=[
        pltpu.VMEM((NG, G), jnp.int32),
        pltpu.VMEM((NBUF, G, NUM_FACTOR), jnp.float32),
        pltpu.SemaphoreType.DMA((NBUF,)),
    ],
    compiler_params=pltpu.CompilerParams(use_tc_tiling_on_sc=False),
)
def _gather_kernel(table_hbm, idx_hbm, out_hbm, idx_v, rows_v, gsems):
    wid = lax.axis_index("s") * NC + lax.axis_index("c")
    base = wid * PER_W

    pltpu.sync_copy(idx_hbm.at[wid], idx_v)

    def fire(g, b):
        pltpu.async_copy(table_hbm.at[idx_v.at[g]], rows_v.at[b], gsems.at[b])

    def wait_store(g, b):
        pltpu.make_async_copy(
            table_hbm.at[idx_v.at[g]], rows_v.at[b], gsems.at[b]).wait()
        pltpu.sync_copy(rows_v.at[b], out_hbm.at[pl.ds(base + g * G, G)])

    for b in range(NBUF):
        fire(b, b)

    def outer(s, _):
        for b in range(NBUF):
            g = s * NBUF + b
            wait_store(g, b)
            fire(g + NBUF, b)
        return _

    lax.fori_loop(0, NSTEP - 1, outer, None)

    for b in range(NBUF):
        wait_store((NSTEP - 1) * NBUF + b, b)


def kernel(target_item_batch, neg_item_batch, W2):
    target = target_item_batch.reshape(BATCH, TARGET_LEN)
    neg = neg_item_batch.reshape(BATCH, NEG_LEN)
    item_batch = jnp.concatenate([target, neg], axis=1)
    idx = item_batch.reshape(NW, NG, G).astype(jnp.int32)
    flat = _gather_kernel(W2, idx)
    item_embedding = flat.reshape(BATCH, SEQ, NUM_FACTOR)
    return (item_batch, item_embedding)
